# baseline (speedup 1.0000x reference)
import functools

import jax
import jax.numpy as jnp
from jax import lax
from jax.experimental import pallas as pl
from jax.experimental.pallas import tpu as pltpu
from jax.experimental.pallas import tpu_sc as plsc

_LAM_U = 0.01
_D = 32
_L = 16
_CHUNK = 128
_MV_W = 65536

_SC_B0 = 11
_SC_NB = 4
_SC_C0 = _SC_B0 * _MV_W
_SC_COLS = _SC_NB * _MV_W
_SC_C1 = _SC_C0 + _SC_COLS
_MV_CHUNK = 2048


def _matvec_body(t_ref, u_ref, o_ref, r_ref):
    i = pl.program_id(0)
    x = t_ref[...]
    u = u_ref[...]
    o_ref[...] = jnp.sum(x * u, axis=0)

    @pl.when(i == 0)
    def _reg():
        r_ref[0, 0] = _LAM_U * jnp.sqrt(jnp.sum(u * u))


@functools.cache
def _matvec_fn(V: int, nsc: int):
    grid = (V + _MV_W - 1) // _MV_W - nsc

    def cmap(i):
        return jnp.where(i < _SC_B0, i, i + nsc)

    return pl.pallas_call(
        _matvec_body,
        grid=(grid,),
        in_specs=[
            pl.BlockSpec((_D, _MV_W), lambda i: (0, cmap(i))),
            pl.BlockSpec((_D, 1), lambda i: (0, 0)),
        ],
        out_specs=[
            pl.BlockSpec((_MV_W,), lambda i: (cmap(i),)),
            pl.BlockSpec(memory_space=pltpu.SMEM),
        ],
        out_shape=[
            jax.ShapeDtypeStruct((V,), jnp.float32),
            jax.ShapeDtypeStruct((1, 1), jnp.float32),
        ],
    )


@functools.cache
def _sc_matvec_fn(V: int, NC: int, NS: int):
    NW = NC * NS
    c_per_w = _SC_COLS // NW
    n_oct = _D // 8
    n_ch = c_per_w // _MV_CHUNK
    n_steps = n_oct * n_ch
    mesh = plsc.VectorSubcoreMesh(core_axis_name="c", subcore_axis_name="s")

    @functools.partial(
        pl.kernel,
        mesh=mesh,
        compiler_params=pltpu.CompilerParams(use_tc_tiling_on_sc=True),
        out_type=jax.ShapeDtypeStruct((_SC_COLS,), jnp.float32),
        scratch_types=[
            pltpu.VMEM((8, _MV_CHUNK), jnp.float32),
            pltpu.VMEM((8, _MV_CHUNK), jnp.float32),
            pltpu.VMEM((_D, _L), jnp.float32),
            pltpu.VMEM((c_per_w,), jnp.float32),
            pltpu.SemaphoreType.DMA,
        ],
    )
    def sc_matvec(tview_hbm, ub_hbm, out_hbm, buf0, buf1, ub_v, acc_v, sem):
        cid = lax.axis_index("c")
        sid = lax.axis_index("s")
        wid = sid * NC + cid
        col0 = _SC_C0 + wid * c_per_w
        pltpu.sync_copy(ub_hbm, ub_v)
        bufs = (buf0, buf1)

        def src(k):
            t, c = k // n_ch, k % n_ch
            return tview_hbm.at[pl.ds(t * 8, 8),
                                pl.ds(col0 + c * _MV_CHUNK, _MV_CHUNK)]

        def compute(k, buf):
            t = k // n_ch
            cbase = (k % n_ch) * _MV_CHUNK

            def body(g, carry):
                o = cbase + g * _L
                a = acc_v[pl.ds(o, _L)]
                for r in range(8):
                    a = a + buf[r, pl.ds(g * _L, _L)] * ub_v[t * 8 + r, :]
                acc_v[pl.ds(o, _L)] = a
                return carry

            lax.fori_loop(0, _MV_CHUNK // _L, body, 0)

        def zbody(g, carry):
            acc_v[pl.ds(g * _L, _L)] = jnp.zeros((_L,), jnp.float32)
            return carry

        lax.fori_loop(0, c_per_w // _L, zbody, 0)

        cps = [pltpu.async_copy(src(0), bufs[0], sem)]
        for k in range(n_steps):
            if k + 1 < n_steps:
                cps.append(pltpu.async_copy(src(k + 1), bufs[(k + 1) % 2], sem))
            cps[k].wait()
            compute(k, bufs[k % 2])
        pltpu.sync_copy(acc_v, out_hbm.at[pl.ds(wid * c_per_w, c_per_w)])

    return sc_matvec


def _lane_sum(v):
    lanes = lax.iota(jnp.int32, _L)
    dnums = lax.GatherDimensionNumbers(
        offset_dims=(), collapsed_slice_dims=(0,), start_index_map=(0,))
    for k in (8, 4, 2, 1):
        idx = lax.bitwise_xor(lanes, jnp.full((_L,), k, jnp.int32))
        v = v + lax.gather(v, idx[:, None], dnums, (1,),
                           mode=lax.GatherScatterMode.PROMISE_IN_BOUNDS)
    return v


def _log1p_exp(t):
    e = jnp.exp(t)
    w = e * (1.0 - e * (0.5 - e * (1.0 / 3.0)))
    for _ in range(3):
        w = w - 1.0 + (1.0 + e) * jnp.exp(-w)
    return w


@functools.cache
def _sc_bce_fn(B: int, NC: int, NS: int):
    NW = NC * NS
    b_per_w = B // NW
    n_chunks = b_per_w // _CHUNK
    mesh = plsc.VectorSubcoreMesh(core_axis_name="c", subcore_axis_name="s")

    @functools.partial(
        pl.kernel,
        mesh=mesh,
        compiler_params=pltpu.CompilerParams(use_tc_tiling_on_sc=False),
        out_type=jax.ShapeDtypeStruct((NC, _L), jnp.float32),
        scratch_types=[
            pltpu.VMEM((n_chunks, _CHUNK), jnp.int32),
            pltpu.VMEM((n_chunks, _CHUNK), jnp.int32),
            pltpu.VMEM((b_per_w,), jnp.float32),
            pltpu.VMEM((b_per_w,), jnp.float32),
            pltpu.VMEM((b_per_w,), jnp.float32),
            pltpu.VMEM((_L,), jnp.float32),
            pltpu.VMEM((NS, _L), jnp.float32),
            pltpu.VMEM_SHARED((NS, _L), jnp.float32),
            pltpu.SemaphoreType.DMA,
        ],
    )
    def sc_bce(item_hbm, y_hbm, ltc_hbm, lsc_hbm, out_hbm,
               idx_v, idx2_v, ga_v, gb_v, y_v, acc_v, stage_v, shared_v, sem):
        cid = lax.axis_index("c")
        sid = lax.axis_index("s")
        wid = sid * NC + cid
        pltpu.sync_copy(item_hbm.at[wid], idx_v)
        pltpu.sync_copy(y_hbm.at[wid], y_v)

        def ibody(g, carry):
            r, c = g // (_CHUNK // _L), g % (_CHUNK // _L)
            iv = idx_v[r, pl.ds(c * _L, _L)]
            off = jnp.clip(iv - _SC_C0, 0, _SC_COLS - 1)
            idx2_v[r, pl.ds(c * _L, _L)] = off
            return carry

        lax.fori_loop(0, b_per_w // _L, ibody, 0)

        copies = []
        for j in range(n_chunks):
            copies.append(pltpu.async_copy(
                ltc_hbm.at[idx_v.at[j]],
                ga_v.at[pl.ds(j * _CHUNK, _CHUNK)],
                sem))
            copies.append(pltpu.async_copy(
                lsc_hbm.at[idx2_v.at[j]],
                gb_v.at[pl.ds(j * _CHUNK, _CHUNK)],
                sem))
        for c in copies:
            c.wait()

        def body(g, acc):
            r, c = g // (_CHUNK // _L), g % (_CHUNK // _L)
            iv = idx_v[r, pl.ds(c * _L, _L)]
            a = ga_v[pl.ds(g * _L, _L)]
            b = gb_v[pl.ds(g * _L, _L)]
            in_sc = (iv >= _SC_C0) & (iv < _SC_C1)
            x = jnp.where(in_sc, b, a)
            y = y_v[pl.ds(g * _L, _L)]
            return acc + jnp.maximum(x, 0.0) - x * y + _log1p_exp(-jnp.abs(x))

        acc = lax.fori_loop(0, b_per_w // _L, body,
                            jnp.zeros((_L,), jnp.float32))
        acc_v[...] = acc
        pltpu.sync_copy(acc_v, shared_v.at[sid])
        plsc.subcore_barrier()

        @pl.when(sid == 0)
        def _reduce():
            pltpu.sync_copy(shared_v, stage_v)
            tot = jnp.zeros((_L,), jnp.float32)
            for s in range(NS):
                tot = tot + stage_v[s, :]
            acc_v[...] = _lane_sum(tot)
            pltpu.sync_copy(acc_v, out_hbm.at[cid])

    return sc_bce


def kernel(item, matrix, user_embeddings, item_embeddings):
    B = item.shape[0]
    V = item_embeddings.shape[0]
    try:
        info = plsc.get_sparse_core_info()
        NC, NS = info.num_cores, info.num_subcores
    except Exception:
        NC, NS = 2, 16
    NW = NC * NS
    b_per_w = B // NW
    n_chunks = b_per_w // _CHUNK

    tview = item_embeddings.T
    u_col = user_embeddings.reshape(_D, 1).astype(jnp.float32)
    ub = jnp.broadcast_to(
        user_embeddings.reshape(_D, 1).astype(jnp.float32), (_D, _L))

    logits_tc, reg = _matvec_fn(V, _SC_NB)(tview, u_col)
    logits_sc = _sc_matvec_fn(V, NC, NS)(tview, ub)

    item_r = item.astype(jnp.int32).reshape(NW, n_chunks, _CHUNK)
    y_r = matrix.astype(jnp.float32).reshape(NW, b_per_w)
    parts = _sc_bce_fn(B, NC, NS)(item_r, y_r, logits_tc, logits_sc)

    return parts[:, 0].sum() + reg[0, 0]


# speedup vs baseline: 1.9563x; 1.9563x over previous
import functools

import jax
import jax.numpy as jnp
from jax import lax
from jax.experimental import pallas as pl
from jax.experimental.pallas import tpu as pltpu
from jax.experimental.pallas import tpu_sc as plsc

_LAM_U = 0.01
_D = 32
_CHUNK = 128
_MV_W = 65536


def _matvec_body(t_ref, u_ref, o_ref):
    x = t_ref[...]
    u = u_ref[...]
    o_ref[...] = jnp.sum(x * u, axis=0)


@functools.cache
def _matvec_fn(V: int):
    grid = (V + _MV_W - 1) // _MV_W
    return pl.pallas_call(
        _matvec_body,
        grid=(grid,),
        in_specs=[
            pl.BlockSpec((_D, _MV_W), lambda i: (0, i)),
            pl.BlockSpec((_D, 1), lambda i: (0, 0)),
        ],
        out_specs=pl.BlockSpec((_MV_W,), lambda i: (i,)),
        out_shape=jax.ShapeDtypeStruct((V,), jnp.float32),
    )


@functools.cache
def _sc_gather_fn(B: int, V: int, NC: int, NS: int):
    NW = NC * NS
    b_per_w = B // NW
    n_chunks = b_per_w // _CHUNK
    mesh = plsc.VectorSubcoreMesh(core_axis_name="c", subcore_axis_name="s")

    @functools.partial(
        pl.kernel,
        mesh=mesh,
        compiler_params=pltpu.CompilerParams(use_tc_tiling_on_sc=False),
        out_type=jax.ShapeDtypeStruct((B,), jnp.float32),
        scratch_types=[
            pltpu.VMEM((n_chunks, _CHUNK), jnp.int32),
            pltpu.VMEM((b_per_w,), jnp.float32),
            pltpu.SemaphoreType.DMA,
        ],
    )
    def sc_gather(item_hbm, logits_hbm, out_hbm, idx_v, g_v, sem):
        wid = lax.axis_index("s") * NC + lax.axis_index("c")
        base = wid * b_per_w
        pltpu.sync_copy(item_hbm.at[wid], idx_v)
        copies = []
        for j in range(n_chunks):
            copies.append(pltpu.async_copy(
                logits_hbm.at[idx_v.at[j]],
                g_v.at[pl.ds(j * _CHUNK, _CHUNK)],
                sem))
        for c in copies:
            c.wait()
        pltpu.sync_copy(g_v, out_hbm.at[pl.ds(base, b_per_w)])

    return sc_gather


def _tc_loss_body(x_ref, y_ref, u_ref, o_ref):
    x = x_ref[...]
    y = y_ref[...]
    bce = jnp.maximum(x, 0.0) - x * y + jnp.log1p(jnp.exp(-jnp.abs(x)))
    u = u_ref[...]
    o_ref[0, 0] = jnp.sum(bce) + _LAM_U * jnp.sqrt(jnp.sum(u * u))


def _tc_loss(logits2d, y2d, u):
    return pl.pallas_call(
        _tc_loss_body,
        out_shape=jax.ShapeDtypeStruct((1, 1), jnp.float32),
        out_specs=pl.BlockSpec(memory_space=pltpu.SMEM),
    )(logits2d, y2d, u)


def kernel(item, matrix, user_embeddings, item_embeddings):
    B = item.shape[0]
    V = item_embeddings.shape[0]
    try:
        info = plsc.get_sparse_core_info()
        NC, NS = info.num_cores, info.num_subcores
    except Exception:
        NC, NS = 2, 16
    NW = NC * NS
    b_per_w = B // NW
    n_chunks = b_per_w // _CHUNK

    tview = item_embeddings.T
    u_col = user_embeddings.reshape(_D, 1).astype(jnp.float32)
    logits_all = _matvec_fn(V)(tview, u_col)

    item_r = item.astype(jnp.int32).reshape(NW, n_chunks, _CHUNK)
    logits = _sc_gather_fn(B, V, NC, NS)(item_r, logits_all)

    u = user_embeddings.reshape(1, _D).astype(jnp.float32)
    out = _tc_loss(logits.reshape(128, 128), matrix.reshape(128, 128), u)
    return out[0, 0]
